# baseline (speedup 1.0000x reference)
import functools

import jax
import jax.numpy as jnp
from jax import lax
from jax.experimental import pallas as pl
from jax.experimental.pallas import tpu as pltpu
from jax.experimental.pallas import tpu_sc as plsc

N_NODES = 10000
NPAD = 10240
STRIPE = NPAD // 16
E_PAD = 163840
CHUNKS = 80
K = 128
CCHUNKS = CHUNKS // 2
NC, NS = 2, 16


def _seg_sum_sc(table, gidx, ridx, z128):
    mesh = plsc.VectorSubcoreMesh(core_axis_name="c", subcore_axis_name="s")

    half = CHUNKS // 2

    @functools.partial(
        pl.kernel,
        out_type=jax.ShapeDtypeStruct((2 * NPAD, 128), jnp.float32),
        mesh=mesh,
        scratch_types=[
            pltpu.VMEM_SHARED((NPAD, 128), jnp.float32),
            pltpu.VMEM((half, K), jnp.int32),
            pltpu.VMEM((half, K), jnp.int32),
            pltpu.VMEM((K, 128), jnp.float32),
            pltpu.VMEM((K, 128), jnp.float32),
            pltpu.SemaphoreType.DMA,
            pltpu.SemaphoreType.DMA,
        ],
    )
    def seg_kernel(table_h, gidx_h, ridx_h, z128_h, out_sum,
                   acc, gi_v, ri_v, data0, data1, sem_g, sem_s):
        c = lax.axis_index("c")
        s = lax.axis_index("s")
        pltpu.sync_copy(z128_h, acc.at[pl.ds(s * STRIPE, STRIPE)])
        plsc.subcore_barrier()

        def g_wait(d):
            pltpu.make_async_copy(table_h.at[gi_v.at[0]], d, sem_g).wait()

        def s_wait(d):
            pltpu.make_async_copy(d, acc.at[gi_v.at[0]], sem_s).wait()

        for p in range(2):
            pltpu.sync_copy(
                gidx_h.at[pl.ds(((c * NS + s) * 2 + p) * half, half)], gi_v)
            pltpu.sync_copy(
                ridx_h.at[pl.ds((s * 2 + p) * half, half)], ri_v)
            pltpu.async_copy(table_h.at[gi_v.at[0]], data0, sem_g)
            pltpu.async_copy(table_h.at[gi_v.at[1]], data1, sem_g)

            def pair(i, carry):
                j = 2 * i
                g_wait(data0)
                pltpu.async_copy(data0, acc.at[ri_v.at[j]], sem_s, add=True)
                g_wait(data1)
                pltpu.async_copy(data1, acc.at[ri_v.at[j + 1]], sem_s,
                                 add=True)

                @pl.when(j + 2 < half)
                def _():
                    s_wait(data0)
                    pltpu.async_copy(table_h.at[gi_v.at[j + 2]], data0,
                                     sem_g)

                @pl.when(j + 3 < half)
                def _():
                    s_wait(data1)
                    pltpu.async_copy(table_h.at[gi_v.at[j + 3]], data1,
                                     sem_g)

                return carry

            lax.fori_loop(0, half // 2, pair, 0)
            s_wait(data0)
            s_wait(data1)
        plsc.subcore_barrier()

        pltpu.sync_copy(acc.at[pl.ds(s * STRIPE, STRIPE)],
                        out_sum.at[pl.ds(c * NPAD + s * STRIPE, STRIPE)])

    return seg_kernel(table, gidx, ridx, z128)


def _seg_cnt_sc(ridx, z128, ones_h):
    mesh = plsc.VectorSubcoreMesh(core_axis_name="c", subcore_axis_name="s")

    @functools.partial(
        pl.kernel,
        out_type=jax.ShapeDtypeStruct((2 * NPAD, 128), jnp.float32),
        mesh=mesh,
        scratch_types=[
            pltpu.VMEM_SHARED((NPAD, 128), jnp.float32),
            pltpu.VMEM((CCHUNKS, K), jnp.int32),
            pltpu.VMEM((K, 128), jnp.float32),
        ],
    )
    def cnt_kernel(ridx_h, z128_h, ones_hbm, out_cnt, cacc, ri_v, ones_v):
        c = lax.axis_index("c")
        s = lax.axis_index("s")
        pltpu.sync_copy(z128_h, cacc.at[pl.ds(s * STRIPE, STRIPE)])
        pltpu.sync_copy(ones_hbm, ones_v)
        pltpu.sync_copy(ridx_h.at[pl.ds((c * NS + s) * CCHUNKS, CCHUNKS)],
                        ri_v)
        plsc.subcore_barrier()

        def chunk(j, carry):
            pltpu.sync_copy(ones_v, cacc.at[ri_v.at[j]], add=True)
            return carry

        lax.fori_loop(0, CCHUNKS, chunk, 0)
        plsc.subcore_barrier()
        pltpu.sync_copy(cacc.at[pl.ds(s * STRIPE, STRIPE)],
                        out_cnt.at[pl.ds(c * NPAD + s * STRIPE, STRIPE)])

    return cnt_kernel(ridx, z128, ones_h)


def _prep_edges(edge_index, n):
    e = edge_index.shape[1]
    order = jnp.argsort(edge_index[1])
    row = edge_index[0][order]
    col = edge_index[1][order]
    pad = E_PAD - e
    row_p = jnp.concatenate([row, jnp.full((pad,), n, jnp.int32)])
    col_p = jnp.concatenate([col, jnp.zeros((pad,), jnp.int32)])
    gidx = jnp.stack([col_p, col_p + n]).reshape(2 * NS * CHUNKS, K)
    ridx = row_p.reshape(NS * CHUNKS, K)
    return gidx, ridx


ROWS = 512
GRID = (N_NODES + ROWS - 1) // ROWS


def _stage_a_body(x_ref, s0_ref, s1_ref, c0_ref, c1_ref,
                  ws_ref, wn0_ref, wn1_ref, b_ref, wn2_ref,
                  h1_ref, t_ref, cnt_ref):
    cnt = c0_ref[0, :, :16] + c1_ref[0, :, :16]
    cnt_ref[...] = cnt
    r = 1.0 / (cnt[:, :1] + 1e-12)
    n0 = s0_ref[0] * r
    n1 = s1_ref[0] * r
    h = (jnp.dot(x_ref[...], ws_ref[...], preferred_element_type=jnp.float32)
         + jnp.dot(n0, wn0_ref[...], preferred_element_type=jnp.float32)
         + jnp.dot(n1, wn1_ref[...], preferred_element_type=jnp.float32)
         + b_ref[...])
    h = jnp.maximum(h, 0.0)
    h1_ref[...] = h
    t = jnp.dot(h, wn2_ref[...], preferred_element_type=jnp.float32)
    t_ref[0] = t[:, :128]
    t_ref[1] = t[:, 128:]


def _stage_a(x, s01, cnt01, ws1, wn1, b1, wn2):
    s01r = s01.reshape(2, NPAD, 128)
    c01r = cnt01.reshape(2, NPAD, 128)
    return pl.pallas_call(
        _stage_a_body,
        grid=(GRID,),
        in_specs=[
            pl.BlockSpec((ROWS, 256), lambda i: (i, 0)),
            pl.BlockSpec((1, ROWS, 128), lambda i: (0, i, 0)),
            pl.BlockSpec((1, ROWS, 128), lambda i: (1, i, 0)),
            pl.BlockSpec((1, ROWS, 128), lambda i: (0, i, 0)),
            pl.BlockSpec((1, ROWS, 128), lambda i: (1, i, 0)),
            pl.BlockSpec((256, 512), lambda i: (0, 0)),
            pl.BlockSpec((128, 512), lambda i: (0, 0)),
            pl.BlockSpec((128, 512), lambda i: (0, 0)),
            pl.BlockSpec((1, 512), lambda i: (0, 0)),
            pl.BlockSpec((512, 256), lambda i: (0, 0)),
        ],
        out_specs=[
            pl.BlockSpec((ROWS, 512), lambda i: (i, 0)),
            pl.BlockSpec((2, ROWS, 128), lambda i: (0, i, 0)),
            pl.BlockSpec((ROWS, 16), lambda i: (i, 0)),
        ],
        out_shape=[
            jax.ShapeDtypeStruct((N_NODES, 512), jnp.float32),
            jax.ShapeDtypeStruct((2, N_NODES, 128), jnp.float32),
            jax.ShapeDtypeStruct((NPAD, 16), jnp.float32),
        ],
    )(x, s01r, s01r, c01r, c01r, ws1, wn1[:128], wn1[128:], b1, wn2)


def _stage_b_body(h1s_ref, ss0_ref, ss1_ref, cs_ref,
                  h1a_ref, sa0_ref, sa1_ref, ca_ref,
                  ws2s_ref, b2s_ref, ws2a_ref, b2a_ref,
                  g1t_ref, g1b_ref, g1bias_ref, g2w_ref, g2b_ref,
                  out_ref):
    rs = 1.0 / (cs_ref[:, :1] + 1e-12)
    ra = 1.0 / (ca_ref[:, :1] + 1e-12)
    nei_s = jnp.concatenate([ss0_ref[0], ss1_ref[0]], axis=1) * rs
    nei_a = jnp.concatenate([sa0_ref[0], sa1_ref[0]], axis=1) * ra
    hs = jnp.maximum(
        jnp.dot(h1s_ref[...], ws2s_ref[...],
                preferred_element_type=jnp.float32) + nei_s + b2s_ref[...],
        0.0)
    ha = jnp.maximum(
        jnp.dot(h1a_ref[...], ws2a_ref[...],
                preferred_element_type=jnp.float32) + nei_a + b2a_ref[...],
        0.0)
    q = jnp.maximum(
        jnp.dot(hs, g1t_ref[...], preferred_element_type=jnp.float32)
        + jnp.dot(ha, g1b_ref[...], preferred_element_type=jnp.float32)
        + g1bias_ref[...],
        0.0)
    g = jnp.sum(q * g2w_ref[...], axis=1, keepdims=True) + g2b_ref[...]
    w = jax.nn.sigmoid(g)
    out_ref[...] = w * hs + (1.0 - w) * ha


def _stage_b(h1s, ss01, cnt_s, h1a, sa01, cnt_a,
             ws2s, b2s, ws2a, b2a, g1w, g1b, g2w, g2b):
    ss = ss01.reshape(2, NPAD, 128)
    sa = sa01.reshape(2, NPAD, 128)
    return pl.pallas_call(
        _stage_b_body,
        grid=(GRID,),
        in_specs=[
            pl.BlockSpec((ROWS, 512), lambda i: (i, 0)),
            pl.BlockSpec((1, ROWS, 128), lambda i: (0, i, 0)),
            pl.BlockSpec((1, ROWS, 128), lambda i: (1, i, 0)),
            pl.BlockSpec((ROWS, 16), lambda i: (i, 0)),
            pl.BlockSpec((ROWS, 512), lambda i: (i, 0)),
            pl.BlockSpec((1, ROWS, 128), lambda i: (0, i, 0)),
            pl.BlockSpec((1, ROWS, 128), lambda i: (1, i, 0)),
            pl.BlockSpec((ROWS, 16), lambda i: (i, 0)),
            pl.BlockSpec((512, 256), lambda i: (0, 0)),
            pl.BlockSpec((1, 256), lambda i: (0, 0)),
            pl.BlockSpec((512, 256), lambda i: (0, 0)),
            pl.BlockSpec((1, 256), lambda i: (0, 0)),
            pl.BlockSpec((256, 256), lambda i: (0, 0)),
            pl.BlockSpec((256, 256), lambda i: (0, 0)),
            pl.BlockSpec((1, 256), lambda i: (0, 0)),
            pl.BlockSpec((1, 256), lambda i: (0, 0)),
            pl.BlockSpec((1, 1), lambda i: (0, 0)),
        ],
        out_specs=pl.BlockSpec((ROWS, 256), lambda i: (i, 0)),
        out_shape=jax.ShapeDtypeStruct((N_NODES, 256), jnp.float32),
    )(h1s, ss, ss, cnt_s, h1a, sa, sa, cnt_a,
      ws2s, b2s, ws2a, b2a, g1w[:256], g1w[256:], g1b, g2w, g2b)


def kernel(x, edge_spatial, edge_attr, params):
    n = x.shape[0]

    gidx_s, ridx_s = _prep_edges(edge_spatial, n)
    gidx_a, ridx_a = _prep_edges(edge_attr, n)

    x_table = jnp.stack([x[:, :128], x[:, 128:]]).reshape(2 * n, 128)

    z128 = jnp.zeros((STRIPE, 128), jnp.float32)
    ones_h = jnp.ones((K, 128), jnp.float32)

    ps1, ps2 = params["s"]
    pa1, pa2 = params["a"]
    b_s1 = (ps1["bs"] + ps1["bn"]).reshape(1, 512)
    b_a1 = (pa1["bs"] + pa1["bn"]).reshape(1, 512)
    b_s2 = (ps2["bs"] + ps2["bn"]).reshape(1, 256)
    b_a2 = (pa2["bs"] + pa2["bn"]).reshape(1, 256)

    cnt01_s = _seg_cnt_sc(ridx_s, z128, ones_h)
    cnt01_a = _seg_cnt_sc(ridx_a, z128, ones_h)

    sum_s1 = _seg_sum_sc(x_table, gidx_s, ridx_s, z128)
    h1s, t_s, cnt_s = _stage_a(x, sum_s1, cnt01_s, ps1["Ws"], ps1["Wn"],
                               b_s1, ps2["Wn"])

    sum_a1 = _seg_sum_sc(x_table, gidx_a, ridx_a, z128)
    h1a, t_a, cnt_a = _stage_a(x, sum_a1, cnt01_a, pa1["Ws"], pa1["Wn"],
                               b_a1, pa2["Wn"])

    sum_s2 = _seg_sum_sc(t_s.reshape(2 * n, 128), gidx_s, ridx_s, z128)
    sum_a2 = _seg_sum_sc(t_a.reshape(2 * n, 128), gidx_a, ridx_a, z128)

    return _stage_b(h1s, sum_s2, cnt_s, h1a, sum_a2, cnt_a,
                    ps2["Ws"], b_s2, pa2["Ws"], b_a2,
                    params["g1W"], params["g1b"].reshape(1, 256),
                    params["g2W"].reshape(1, 256),
                    params["g2b"].reshape(1, 1))


# speedup vs baseline: 1.4196x; 1.4196x over previous
import functools

import jax
import jax.numpy as jnp
from jax import lax
from jax.experimental import pallas as pl
from jax.experimental.pallas import tpu as pltpu
from jax.experimental.pallas import tpu_sc as plsc

N_NODES = 10000
NPAD = 10240
STRIPE = NPAD // 16
E_PAD = 163840
CHUNKS = 80
K = 128
CCHUNKS = CHUNKS // 2
NC, NS = 2, 16


def _seg_sum_sc(table, gidx, ridx, z128):
    mesh = plsc.VectorSubcoreMesh(core_axis_name="c", subcore_axis_name="s")

    half = CHUNKS // 2

    @functools.partial(
        pl.kernel,
        out_type=jax.ShapeDtypeStruct((2 * NPAD, 128), jnp.float32),
        mesh=mesh,
        scratch_types=[
            pltpu.VMEM_SHARED((NPAD, 128), jnp.float32),
            pltpu.VMEM((half, K), jnp.int32),
            pltpu.VMEM((half, K), jnp.int32),
            pltpu.VMEM((K, 128), jnp.float32),
            pltpu.VMEM((K, 128), jnp.float32),
            pltpu.SemaphoreType.DMA,
            pltpu.SemaphoreType.DMA,
        ],
    )
    def seg_kernel(table_h, gidx_h, ridx_h, z128_h, out_sum,
                   acc, gi_v, ri_v, data0, data1, sem_g, sem_s):
        c = lax.axis_index("c")
        s = lax.axis_index("s")
        pltpu.sync_copy(z128_h, acc.at[pl.ds(s * STRIPE, STRIPE)])
        plsc.subcore_barrier()

        def g_wait(d):
            pltpu.make_async_copy(table_h.at[gi_v.at[0]], d, sem_g).wait()

        def s_wait(d):
            pltpu.make_async_copy(d, acc.at[gi_v.at[0]], sem_s).wait()

        for p in range(2):
            pltpu.sync_copy(
                gidx_h.at[pl.ds(((c * NS + s) * 2 + p) * half, half)], gi_v)
            pltpu.sync_copy(
                ridx_h.at[pl.ds((s * 2 + p) * half, half)], ri_v)
            pltpu.async_copy(table_h.at[gi_v.at[0]], data0, sem_g)
            pltpu.async_copy(table_h.at[gi_v.at[1]], data1, sem_g)

            def pair(i, carry):
                j = 2 * i
                g_wait(data0)
                pltpu.async_copy(data0, acc.at[ri_v.at[j]], sem_s, add=True)
                g_wait(data1)
                pltpu.async_copy(data1, acc.at[ri_v.at[j + 1]], sem_s,
                                 add=True)

                @pl.when(j + 2 < half)
                def _():
                    s_wait(data0)
                    pltpu.async_copy(table_h.at[gi_v.at[j + 2]], data0,
                                     sem_g)

                @pl.when(j + 3 < half)
                def _():
                    s_wait(data1)
                    pltpu.async_copy(table_h.at[gi_v.at[j + 3]], data1,
                                     sem_g)

                return carry

            lax.fori_loop(0, half // 2, pair, 0)
            s_wait(data0)
            s_wait(data1)
        plsc.subcore_barrier()

        pltpu.sync_copy(acc.at[pl.ds(s * STRIPE, STRIPE)],
                        out_sum.at[pl.ds(c * NPAD + s * STRIPE, STRIPE)])

    return seg_kernel(table, gidx, ridx, z128)


def _seg_cnt_sc(ridx, z128, ones_h):
    mesh = plsc.VectorSubcoreMesh(core_axis_name="c", subcore_axis_name="s")

    @functools.partial(
        pl.kernel,
        out_type=jax.ShapeDtypeStruct((2 * NPAD, 128), jnp.float32),
        mesh=mesh,
        scratch_types=[
            pltpu.VMEM_SHARED((NPAD, 128), jnp.float32),
            pltpu.VMEM((CCHUNKS, K), jnp.int32),
            pltpu.VMEM((K, 128), jnp.float32),
        ],
    )
    def cnt_kernel(ridx_h, z128_h, ones_hbm, out_cnt, cacc, ri_v, ones_v):
        c = lax.axis_index("c")
        s = lax.axis_index("s")
        pltpu.sync_copy(z128_h, cacc.at[pl.ds(s * STRIPE, STRIPE)])
        pltpu.sync_copy(ones_hbm, ones_v)
        pltpu.sync_copy(ridx_h.at[pl.ds((c * NS + s) * CCHUNKS, CCHUNKS)],
                        ri_v)
        plsc.subcore_barrier()

        def chunk(j, carry):
            pltpu.sync_copy(ones_v, cacc.at[ri_v.at[j]], add=True)
            return carry

        lax.fori_loop(0, CCHUNKS, chunk, 0)
        plsc.subcore_barrier()
        pltpu.sync_copy(cacc.at[pl.ds(s * STRIPE, STRIPE)],
                        out_cnt.at[pl.ds(c * NPAD + s * STRIPE, STRIPE)])

    return cnt_kernel(ridx, z128, ones_h)


def _prep_edges(edge_index, n):
    e = edge_index.shape[1]
    row = edge_index[0]
    col = edge_index[1]
    pad = E_PAD - e
    row_p = jnp.concatenate([row, jnp.full((pad,), n, jnp.int32)])
    col_p = jnp.concatenate([col, jnp.zeros((pad,), jnp.int32)])
    gidx = jnp.stack([2 * col_p, 2 * col_p + 1]).reshape(2 * NS * CHUNKS, K)
    ridx = row_p.reshape(NS * CHUNKS, K)
    return gidx, ridx


ROWS = 512
GRID = (N_NODES + ROWS - 1) // ROWS


def _stage_a_body(x_ref, s0_ref, s1_ref, c0_ref, c1_ref,
                  ws_ref, wn0_ref, wn1_ref, b_ref, wn2_ref,
                  h1_ref, t_ref, cnt_ref):
    cnt = c0_ref[0, :, :16] + c1_ref[0, :, :16]
    cnt_ref[...] = cnt
    r = 1.0 / (cnt[:, :1] + 1e-12)
    n0 = s0_ref[0] * r
    n1 = s1_ref[0] * r
    h = (jnp.dot(x_ref[...], ws_ref[...], preferred_element_type=jnp.float32)
         + jnp.dot(n0, wn0_ref[...], preferred_element_type=jnp.float32)
         + jnp.dot(n1, wn1_ref[...], preferred_element_type=jnp.float32)
         + b_ref[...])
    h = jnp.maximum(h, 0.0)
    h1_ref[...] = h
    t_ref[...] = jnp.dot(h, wn2_ref[...], preferred_element_type=jnp.float32)


def _stage_a(x, s01, cnt01, ws1, wn1, b1, wn2):
    s01r = s01.reshape(2, NPAD, 128)
    c01r = cnt01.reshape(2, NPAD, 128)
    return pl.pallas_call(
        _stage_a_body,
        grid=(GRID,),
        in_specs=[
            pl.BlockSpec((ROWS, 256), lambda i: (i, 0)),
            pl.BlockSpec((1, ROWS, 128), lambda i: (0, i, 0)),
            pl.BlockSpec((1, ROWS, 128), lambda i: (1, i, 0)),
            pl.BlockSpec((1, ROWS, 128), lambda i: (0, i, 0)),
            pl.BlockSpec((1, ROWS, 128), lambda i: (1, i, 0)),
            pl.BlockSpec((256, 512), lambda i: (0, 0)),
            pl.BlockSpec((128, 512), lambda i: (0, 0)),
            pl.BlockSpec((128, 512), lambda i: (0, 0)),
            pl.BlockSpec((1, 512), lambda i: (0, 0)),
            pl.BlockSpec((512, 256), lambda i: (0, 0)),
        ],
        out_specs=[
            pl.BlockSpec((ROWS, 512), lambda i: (i, 0)),
            pl.BlockSpec((ROWS, 256), lambda i: (i, 0)),
            pl.BlockSpec((ROWS, 16), lambda i: (i, 0)),
        ],
        out_shape=[
            jax.ShapeDtypeStruct((N_NODES, 512), jnp.float32),
            jax.ShapeDtypeStruct((N_NODES, 256), jnp.float32),
            jax.ShapeDtypeStruct((NPAD, 16), jnp.float32),
        ],
    )(x, s01r, s01r, c01r, c01r, ws1, wn1[:128], wn1[128:], b1, wn2)


def _stage_b_body(h1s_ref, ss0_ref, ss1_ref, cs_ref,
                  h1a_ref, sa0_ref, sa1_ref, ca_ref,
                  ws2s_ref, b2s_ref, ws2a_ref, b2a_ref,
                  g1t_ref, g1b_ref, g1bias_ref, g2w_ref, g2b_ref,
                  out_ref):
    rs = 1.0 / (cs_ref[:, :1] + 1e-12)
    ra = 1.0 / (ca_ref[:, :1] + 1e-12)
    nei_s = jnp.concatenate([ss0_ref[0], ss1_ref[0]], axis=1) * rs
    nei_a = jnp.concatenate([sa0_ref[0], sa1_ref[0]], axis=1) * ra
    hs = jnp.maximum(
        jnp.dot(h1s_ref[...], ws2s_ref[...],
                preferred_element_type=jnp.float32) + nei_s + b2s_ref[...],
        0.0)
    ha = jnp.maximum(
        jnp.dot(h1a_ref[...], ws2a_ref[...],
                preferred_element_type=jnp.float32) + nei_a + b2a_ref[...],
        0.0)
    q = jnp.maximum(
        jnp.dot(hs, g1t_ref[...], preferred_element_type=jnp.float32)
        + jnp.dot(ha, g1b_ref[...], preferred_element_type=jnp.float32)
        + g1bias_ref[...],
        0.0)
    g = jnp.sum(q * g2w_ref[...], axis=1, keepdims=True) + g2b_ref[...]
    w = jax.nn.sigmoid(g)
    out_ref[...] = w * hs + (1.0 - w) * ha


def _stage_b(h1s, ss01, cnt_s, h1a, sa01, cnt_a,
             ws2s, b2s, ws2a, b2a, g1w, g1b, g2w, g2b):
    ss = ss01.reshape(2, NPAD, 128)
    sa = sa01.reshape(2, NPAD, 128)
    return pl.pallas_call(
        _stage_b_body,
        grid=(GRID,),
        in_specs=[
            pl.BlockSpec((ROWS, 512), lambda i: (i, 0)),
            pl.BlockSpec((1, ROWS, 128), lambda i: (0, i, 0)),
            pl.BlockSpec((1, ROWS, 128), lambda i: (1, i, 0)),
            pl.BlockSpec((ROWS, 16), lambda i: (i, 0)),
            pl.BlockSpec((ROWS, 512), lambda i: (i, 0)),
            pl.BlockSpec((1, ROWS, 128), lambda i: (0, i, 0)),
            pl.BlockSpec((1, ROWS, 128), lambda i: (1, i, 0)),
            pl.BlockSpec((ROWS, 16), lambda i: (i, 0)),
            pl.BlockSpec((512, 256), lambda i: (0, 0)),
            pl.BlockSpec((1, 256), lambda i: (0, 0)),
            pl.BlockSpec((512, 256), lambda i: (0, 0)),
            pl.BlockSpec((1, 256), lambda i: (0, 0)),
            pl.BlockSpec((256, 256), lambda i: (0, 0)),
            pl.BlockSpec((256, 256), lambda i: (0, 0)),
            pl.BlockSpec((1, 256), lambda i: (0, 0)),
            pl.BlockSpec((1, 256), lambda i: (0, 0)),
            pl.BlockSpec((1, 1), lambda i: (0, 0)),
        ],
        out_specs=pl.BlockSpec((ROWS, 256), lambda i: (i, 0)),
        out_shape=jax.ShapeDtypeStruct((N_NODES, 256), jnp.float32),
    )(h1s, ss, ss, cnt_s, h1a, sa, sa, cnt_a,
      ws2s, b2s, ws2a, b2a, g1w[:256], g1w[256:], g1b, g2w, g2b)


def kernel(x, edge_spatial, edge_attr, params):
    n = x.shape[0]

    gidx_s, ridx_s = _prep_edges(edge_spatial, n)
    gidx_a, ridx_a = _prep_edges(edge_attr, n)

    x_table = x.reshape(2 * n, 128)

    z128 = jnp.zeros((STRIPE, 128), jnp.float32)
    ones_h = jnp.ones((K, 128), jnp.float32)

    ps1, ps2 = params["s"]
    pa1, pa2 = params["a"]
    b_s1 = (ps1["bs"] + ps1["bn"]).reshape(1, 512)
    b_a1 = (pa1["bs"] + pa1["bn"]).reshape(1, 512)
    b_s2 = (ps2["bs"] + ps2["bn"]).reshape(1, 256)
    b_a2 = (pa2["bs"] + pa2["bn"]).reshape(1, 256)

    cnt01_s = _seg_cnt_sc(ridx_s, z128, ones_h)
    cnt01_a = _seg_cnt_sc(ridx_a, z128, ones_h)

    sum_s1 = _seg_sum_sc(x_table, gidx_s, ridx_s, z128)
    h1s, t_s, cnt_s = _stage_a(x, sum_s1, cnt01_s, ps1["Ws"], ps1["Wn"],
                               b_s1, ps2["Wn"])

    sum_a1 = _seg_sum_sc(x_table, gidx_a, ridx_a, z128)
    h1a, t_a, cnt_a = _stage_a(x, sum_a1, cnt01_a, pa1["Ws"], pa1["Wn"],
                               b_a1, pa2["Wn"])

    sum_s2 = _seg_sum_sc(t_s.reshape(2 * n, 128), gidx_s, ridx_s, z128)
    sum_a2 = _seg_sum_sc(t_a.reshape(2 * n, 128), gidx_a, ridx_a, z128)

    return _stage_b(h1s, sum_s2, cnt_s, h1a, sum_a2, cnt_a,
                    ps2["Ws"], b_s2, pa2["Ws"], b_a2,
                    params["g1W"], params["g1b"].reshape(1, 256),
                    params["g2W"].reshape(1, 256),
                    params["g2b"].reshape(1, 1))


# speedup vs baseline: 1.4694x; 1.0351x over previous
import functools

import jax
import jax.numpy as jnp
from jax import lax
from jax.experimental import pallas as pl
from jax.experimental.pallas import tpu as pltpu
from jax.experimental.pallas import tpu_sc as plsc

N_NODES = 10000
NPAD = 10240
STRIPE = NPAD // 16
E_PAD = 163840
CHUNKS = 80
K = 128
CCHUNKS = CHUNKS // 2
NC, NS = 2, 16


def _seg_sum_sc(table, gidx, ridx, z128):
    mesh = plsc.VectorSubcoreMesh(core_axis_name="c", subcore_axis_name="s")

    half = CHUNKS // 2

    @functools.partial(
        pl.kernel,
        out_type=jax.ShapeDtypeStruct((2 * NPAD, 128), jnp.float32),
        mesh=mesh,
        scratch_types=[
            pltpu.VMEM_SHARED((NPAD, 128), jnp.float32),
            pltpu.VMEM((half, K), jnp.int32),
            pltpu.VMEM((half, K), jnp.int32),
            pltpu.VMEM((K, 128), jnp.float32),
            pltpu.VMEM((K, 128), jnp.float32),
            pltpu.SemaphoreType.DMA,
            pltpu.SemaphoreType.DMA,
        ],
    )
    def seg_kernel(table_h, gidx_h, ridx_h, z128_h, out_sum,
                   acc, gi_v, ri_v, data0, data1, sem_g, sem_s):
        c = lax.axis_index("c")
        s = lax.axis_index("s")
        pltpu.sync_copy(z128_h, acc.at[pl.ds(s * STRIPE, STRIPE)])
        plsc.subcore_barrier()

        def g_wait(d):
            pltpu.make_async_copy(table_h.at[gi_v.at[0]], d, sem_g).wait()

        def s_wait(d):
            pltpu.make_async_copy(d, acc.at[gi_v.at[0]], sem_s).wait()

        for p in range(2):
            pltpu.sync_copy(
                gidx_h.at[pl.ds(((c * NS + s) * 2 + p) * half, half)], gi_v)
            pltpu.sync_copy(
                ridx_h.at[pl.ds((s * 2 + p) * half, half)], ri_v)
            pltpu.async_copy(table_h.at[gi_v.at[0]], data0, sem_g)
            pltpu.async_copy(table_h.at[gi_v.at[1]], data1, sem_g)

            def pair(i, carry):
                j = 2 * i
                g_wait(data0)
                pltpu.async_copy(data0, acc.at[ri_v.at[j]], sem_s, add=True)
                g_wait(data1)
                pltpu.async_copy(data1, acc.at[ri_v.at[j + 1]], sem_s,
                                 add=True)

                @pl.when(j + 2 < half)
                def _():
                    s_wait(data0)
                    pltpu.async_copy(table_h.at[gi_v.at[j + 2]], data0,
                                     sem_g)

                @pl.when(j + 3 < half)
                def _():
                    s_wait(data1)
                    pltpu.async_copy(table_h.at[gi_v.at[j + 3]], data1,
                                     sem_g)

                return carry

            lax.fori_loop(0, half // 2, pair, 0)
            s_wait(data0)
            s_wait(data1)
        plsc.subcore_barrier()

        pltpu.sync_copy(acc.at[pl.ds(s * STRIPE, STRIPE)],
                        out_sum.at[pl.ds(c * NPAD + s * STRIPE, STRIPE)])

    return seg_kernel(table, gidx, ridx, z128)


def _seg_cnt_sc(ridx, z128, ones_h):
    mesh = plsc.VectorSubcoreMesh(core_axis_name="c", subcore_axis_name="s")

    @functools.partial(
        pl.kernel,
        out_type=jax.ShapeDtypeStruct((2 * NPAD, 128), jnp.float32),
        mesh=mesh,
        scratch_types=[
            pltpu.VMEM_SHARED((NPAD, 128), jnp.float32),
            pltpu.VMEM((CCHUNKS, K), jnp.int32),
            pltpu.VMEM((K, 128), jnp.float32),
        ],
    )
    def cnt_kernel(ridx_h, z128_h, ones_hbm, out_cnt, cacc, ri_v, ones_v):
        c = lax.axis_index("c")
        s = lax.axis_index("s")
        pltpu.sync_copy(z128_h, cacc.at[pl.ds(s * STRIPE, STRIPE)])
        pltpu.sync_copy(ones_hbm, ones_v)
        pltpu.sync_copy(ridx_h.at[pl.ds((c * NS + s) * CCHUNKS, CCHUNKS)],
                        ri_v)
        plsc.subcore_barrier()

        def chunk(j, carry):
            pltpu.sync_copy(ones_v, cacc.at[ri_v.at[j]], add=True)
            return carry

        lax.fori_loop(0, CCHUNKS, chunk, 0)
        plsc.subcore_barrier()
        pltpu.sync_copy(cacc.at[pl.ds(s * STRIPE, STRIPE)],
                        out_cnt.at[pl.ds(c * NPAD + s * STRIPE, STRIPE)])

    return cnt_kernel(ridx, z128, ones_h)


def _prep_edges(edge_index, n):
    e = edge_index.shape[1]
    row = edge_index[0]
    col = edge_index[1]
    pad = E_PAD - e
    row_p = jnp.concatenate([row, jnp.full((pad,), n, jnp.int32)])
    col_p = jnp.concatenate([col, jnp.zeros((pad,), jnp.int32)])
    gidx = jnp.stack([2 * col_p, 2 * col_p + 1]).reshape(2 * NS * CHUNKS, K)
    ridx = row_p.reshape(NS * CHUNKS, K)
    return gidx, ridx


ROWS = 512
GRID = (N_NODES + ROWS - 1) // ROWS


def _stage_a_body(x_ref, s0_ref, s1_ref, c0_ref, c1_ref,
                  ws_ref, wn0_ref, wn1_ref, b_ref, wn2_ref,
                  h1_ref, t_ref, cnt_ref):
    cnt = c0_ref[0, :, :16] + c1_ref[0, :, :16]
    cnt_ref[...] = cnt
    r = 1.0 / (cnt[:, :1] + 1e-12)
    n0 = s0_ref[0] * r
    n1 = s1_ref[0] * r
    bf = jnp.bfloat16
    h = (jnp.dot(x_ref[...].astype(bf), ws_ref[...].astype(bf),
                 preferred_element_type=jnp.float32)
         + jnp.dot(n0.astype(bf), wn0_ref[...].astype(bf),
                   preferred_element_type=jnp.float32)
         + jnp.dot(n1.astype(bf), wn1_ref[...].astype(bf),
                   preferred_element_type=jnp.float32)
         + b_ref[...])
    h = jnp.maximum(h, 0.0)
    h1_ref[...] = h
    t_ref[...] = jnp.dot(h.astype(bf), wn2_ref[...].astype(bf),
                         preferred_element_type=jnp.float32)


def _stage_a(x, s01, cnt01, ws1, wn1, b1, wn2):
    s01r = s01.reshape(2, NPAD, 128)
    c01r = cnt01.reshape(2, NPAD, 128)
    return pl.pallas_call(
        _stage_a_body,
        grid=(GRID,),
        in_specs=[
            pl.BlockSpec((ROWS, 256), lambda i: (i, 0)),
            pl.BlockSpec((1, ROWS, 128), lambda i: (0, i, 0)),
            pl.BlockSpec((1, ROWS, 128), lambda i: (1, i, 0)),
            pl.BlockSpec((1, ROWS, 128), lambda i: (0, i, 0)),
            pl.BlockSpec((1, ROWS, 128), lambda i: (1, i, 0)),
            pl.BlockSpec((256, 512), lambda i: (0, 0)),
            pl.BlockSpec((128, 512), lambda i: (0, 0)),
            pl.BlockSpec((128, 512), lambda i: (0, 0)),
            pl.BlockSpec((1, 512), lambda i: (0, 0)),
            pl.BlockSpec((512, 256), lambda i: (0, 0)),
        ],
        out_specs=[
            pl.BlockSpec((ROWS, 512), lambda i: (i, 0)),
            pl.BlockSpec((ROWS, 256), lambda i: (i, 0)),
            pl.BlockSpec((ROWS, 16), lambda i: (i, 0)),
        ],
        out_shape=[
            jax.ShapeDtypeStruct((N_NODES, 512), jnp.float32),
            jax.ShapeDtypeStruct((N_NODES, 256), jnp.float32),
            jax.ShapeDtypeStruct((NPAD, 16), jnp.float32),
        ],
    )(x, s01r, s01r, c01r, c01r, ws1, wn1[:128], wn1[128:], b1, wn2)


def _stage_b_body(h1s_ref, ss0_ref, ss1_ref, cs_ref,
                  h1a_ref, sa0_ref, sa1_ref, ca_ref,
                  ws2s_ref, b2s_ref, ws2a_ref, b2a_ref,
                  g1t_ref, g1b_ref, g1bias_ref, g2w_ref, g2b_ref,
                  out_ref):
    rs = 1.0 / (cs_ref[:, :1] + 1e-12)
    ra = 1.0 / (ca_ref[:, :1] + 1e-12)
    nei_s = jnp.concatenate([ss0_ref[0], ss1_ref[0]], axis=1) * rs
    nei_a = jnp.concatenate([sa0_ref[0], sa1_ref[0]], axis=1) * ra
    bf = jnp.bfloat16
    hs = jnp.maximum(
        jnp.dot(h1s_ref[...].astype(bf), ws2s_ref[...].astype(bf),
                preferred_element_type=jnp.float32) + nei_s + b2s_ref[...],
        0.0)
    ha = jnp.maximum(
        jnp.dot(h1a_ref[...].astype(bf), ws2a_ref[...].astype(bf),
                preferred_element_type=jnp.float32) + nei_a + b2a_ref[...],
        0.0)
    q = jnp.maximum(
        jnp.dot(hs.astype(bf), g1t_ref[...].astype(bf),
                preferred_element_type=jnp.float32)
        + jnp.dot(ha.astype(bf), g1b_ref[...].astype(bf),
                  preferred_element_type=jnp.float32)
        + g1bias_ref[...],
        0.0)
    g = jnp.sum(q * g2w_ref[...], axis=1, keepdims=True) + g2b_ref[...]
    w = jax.nn.sigmoid(g)
    out_ref[...] = w * hs + (1.0 - w) * ha


def _stage_b(h1s, ss01, cnt_s, h1a, sa01, cnt_a,
             ws2s, b2s, ws2a, b2a, g1w, g1b, g2w, g2b):
    ss = ss01.reshape(2, NPAD, 128)
    sa = sa01.reshape(2, NPAD, 128)
    return pl.pallas_call(
        _stage_b_body,
        grid=(GRID,),
        in_specs=[
            pl.BlockSpec((ROWS, 512), lambda i: (i, 0)),
            pl.BlockSpec((1, ROWS, 128), lambda i: (0, i, 0)),
            pl.BlockSpec((1, ROWS, 128), lambda i: (1, i, 0)),
            pl.BlockSpec((ROWS, 16), lambda i: (i, 0)),
            pl.BlockSpec((ROWS, 512), lambda i: (i, 0)),
            pl.BlockSpec((1, ROWS, 128), lambda i: (0, i, 0)),
            pl.BlockSpec((1, ROWS, 128), lambda i: (1, i, 0)),
            pl.BlockSpec((ROWS, 16), lambda i: (i, 0)),
            pl.BlockSpec((512, 256), lambda i: (0, 0)),
            pl.BlockSpec((1, 256), lambda i: (0, 0)),
            pl.BlockSpec((512, 256), lambda i: (0, 0)),
            pl.BlockSpec((1, 256), lambda i: (0, 0)),
            pl.BlockSpec((256, 256), lambda i: (0, 0)),
            pl.BlockSpec((256, 256), lambda i: (0, 0)),
            pl.BlockSpec((1, 256), lambda i: (0, 0)),
            pl.BlockSpec((1, 256), lambda i: (0, 0)),
            pl.BlockSpec((1, 1), lambda i: (0, 0)),
        ],
        out_specs=pl.BlockSpec((ROWS, 256), lambda i: (i, 0)),
        out_shape=jax.ShapeDtypeStruct((N_NODES, 256), jnp.float32),
    )(h1s, ss, ss, cnt_s, h1a, sa, sa, cnt_a,
      ws2s, b2s, ws2a, b2a, g1w[:256], g1w[256:], g1b, g2w, g2b)


def kernel(x, edge_spatial, edge_attr, params):
    n = x.shape[0]

    gidx_s, ridx_s = _prep_edges(edge_spatial, n)
    gidx_a, ridx_a = _prep_edges(edge_attr, n)

    x_table = x.reshape(2 * n, 128)

    z128 = jnp.zeros((STRIPE, 128), jnp.float32)
    ones_h = jnp.ones((K, 128), jnp.float32)

    ps1, ps2 = params["s"]
    pa1, pa2 = params["a"]
    b_s1 = (ps1["bs"] + ps1["bn"]).reshape(1, 512)
    b_a1 = (pa1["bs"] + pa1["bn"]).reshape(1, 512)
    b_s2 = (ps2["bs"] + ps2["bn"]).reshape(1, 256)
    b_a2 = (pa2["bs"] + pa2["bn"]).reshape(1, 256)

    cnt01_s = _seg_cnt_sc(ridx_s, z128, ones_h)
    cnt01_a = _seg_cnt_sc(ridx_a, z128, ones_h)

    sum_s1 = _seg_sum_sc(x_table, gidx_s, ridx_s, z128)
    h1s, t_s, cnt_s = _stage_a(x, sum_s1, cnt01_s, ps1["Ws"], ps1["Wn"],
                               b_s1, ps2["Wn"])

    sum_a1 = _seg_sum_sc(x_table, gidx_a, ridx_a, z128)
    h1a, t_a, cnt_a = _stage_a(x, sum_a1, cnt01_a, pa1["Ws"], pa1["Wn"],
                               b_a1, pa2["Wn"])

    sum_s2 = _seg_sum_sc(t_s.reshape(2 * n, 128), gidx_s, ridx_s, z128)
    sum_a2 = _seg_sum_sc(t_a.reshape(2 * n, 128), gidx_a, ridx_a, z128)

    return _stage_b(h1s, sum_s2, cnt_s, h1a, sum_a2, cnt_a,
                    ps2["Ws"], b_s2, pa2["Ws"], b_a2,
                    params["g1W"], params["g1b"].reshape(1, 256),
                    params["g2W"].reshape(1, 256),
                    params["g2b"].reshape(1, 1))
